# baseline (speedup 1.0000x reference)
import functools

import jax
import jax.numpy as jnp
import numpy as _np
from jax import lax
from jax.experimental import pallas as pl
from jax.experimental.pallas import tpu as pltpu
from jax.experimental.pallas import tpu_sc as plsc

N = 10000
E = 320000
H = 128
R = 128

LOG2 = 0.6931471805599453

NC = 2
NS = 16
CHUNK = 64
NCHUNK = E // CHUNK
MAX_ITERS = -(-NCHUNK // (NC * NS))
N_PAD = 10240
ROWS_PER_TILE = N_PAD // NS

def _ssp(v):
    return jnp.maximum(v, 0.0) + jnp.log(1.0 + jnp.exp(-jnp.abs(v))) - LOG2


def _h_body(x_ref, w_ref, b_ref, o_ref):
    o_ref[...] = (
        jnp.dot(x_ref[...], w_ref[...], preferred_element_type=jnp.float32)
        + b_ref[...]
    )


def _atomwise1(x, W1, b1):
    blk = 1000
    return pl.pallas_call(
        _h_body,
        grid=(N // blk,),
        in_specs=[
            pl.BlockSpec((blk, H), lambda i: (i, 0)),
            pl.BlockSpec((H, H), lambda i: (0, 0)),
            pl.BlockSpec((1, H), lambda i: (0, 0)),
        ],
        out_specs=pl.BlockSpec((blk, H), lambda i: (i, 0)),
        out_shape=jax.ShapeDtypeStruct((N, H), jnp.float32),
    )(x, W1, b1.reshape(1, H))


def _w_body(rbf_ref, wf1_ref, bf1_ref, wf2_ref, bf2_ref, o_ref):
    t = (
        jnp.dot(rbf_ref[...], wf1_ref[...], preferred_element_type=jnp.float32)
        + bf1_ref[...]
    )
    t = _ssp(t)
    o_ref[...] = (
        jnp.dot(t, wf2_ref[...], preferred_element_type=jnp.float32)
        + bf2_ref[...]
    )


def _filter_net(rbf, Wf1, bf1, Wf2, bf2):
    blk = 2000
    ne = rbf.shape[0]
    return pl.pallas_call(
        _w_body,
        grid=(ne // blk,),
        in_specs=[
            pl.BlockSpec((blk, R), lambda i: (i, 0)),
            pl.BlockSpec((R, H), lambda i: (0, 0)),
            pl.BlockSpec((1, H), lambda i: (0, 0)),
            pl.BlockSpec((H, H), lambda i: (0, 0)),
            pl.BlockSpec((1, H), lambda i: (0, 0)),
        ],
        out_specs=pl.BlockSpec((blk, H), lambda i: (i, 0)),
        out_shape=jax.ShapeDtypeStruct((ne, H), jnp.float32),
    )(rbf, Wf1, bf1.reshape(1, H), Wf2, bf2.reshape(1, H))


def _sc_body(h_hbm, src_hbm, dst_hbm, w_hbm, zeros_hbm, out_hbm,
             sslab, didx0, didx1, hrows0, hrows1, wrows0, wrows1, acc,
             sem_d0, sem_d1, sem_g0, sem_g1, sem_w0, sem_w1,
             nchunk=NCHUNK, max_iters=MAX_ITERS):
    c_id = lax.axis_index("c")
    s_id = lax.axis_index("s")
    g = c_id * NS + s_id
    c0 = (nchunk * g) // (NC * NS)
    cnt = (nchunk * (g + 1)) // (NC * NS) - c0

    bufs = (
        (didx0, hrows0, wrows0, sem_d0, sem_g0, sem_w0),
        (didx1, hrows1, wrows1, sem_d1, sem_g1, sem_w1),
    )

    pltpu.sync_copy(zeros_hbm, acc.at[pl.ds(s_id * ROWS_PER_TILE, ROWS_PER_TILE)])
    pltpu.sync_copy(src_hbm.at[pl.ds(c0 * CHUNK, max_iters * CHUNK)], sslab)
    plsc.subcore_barrier()

    def issue(i, b):
        didx, hrows, wrows, sem_d, sem_g, sem_w = bufs[b]
        pltpu.async_copy(dst_hbm.at[pl.ds((c0 + i) * CHUNK, CHUNK)], didx, sem_d)
        pltpu.async_copy(w_hbm.at[pl.ds((c0 + i) * CHUNK, CHUNK)], wrows, sem_w)
        pltpu.async_copy(h_hbm.at[sslab.at[pl.ds(i * CHUNK, CHUNK)]], hrows, sem_g)

    def wait_issue(i, b):
        didx, hrows, wrows, sem_d, sem_g, sem_w = bufs[b]
        pltpu.make_async_copy(
            dst_hbm.at[pl.ds((c0 + i) * CHUNK, CHUNK)], didx, sem_d).wait()
        pltpu.make_async_copy(
            w_hbm.at[pl.ds((c0 + i) * CHUNK, CHUNK)], wrows, sem_w).wait()
        pltpu.make_async_copy(
            h_hbm.at[sslab.at[pl.ds(i * CHUNK, CHUNK)]], hrows, sem_g).wait()

    issue(0, 0)

    def outer(k, carry):
        for b in range(2):
            i = k * 2 + b
            didx, hrows, wrows, _, _, _ = bufs[b]

            @pl.when(i + 1 < cnt)
            def _():
                issue(i + 1, 1 - b)

            @pl.when(i < cnt)
            def _():
                wait_issue(i, b)

                def mul_row(r, carry2):
                    for q in range(H // 16):
                        sl = pl.ds(q * 16, 16)
                        hrows[r, sl] = hrows[r, sl] * wrows[r, sl]
                    return carry2

                lax.fori_loop(0, CHUNK, mul_row, 0)
                pltpu.sync_copy(hrows, acc.at[didx], add=True)

        return carry

    lax.fori_loop(0, (max_iters + 1) // 2, outer, 0)
    plsc.subcore_barrier()

    pltpu.sync_copy(
        acc.at[pl.ds(s_id * ROWS_PER_TILE, ROWS_PER_TILE)],
        out_hbm.at[c_id, pl.ds(s_id * ROWS_PER_TILE, ROWS_PER_TILE)],
    )


def _cfconv_scatter(h, src_half, dst_half, w, zeros, nchunk, max_iters):
    mesh = plsc.VectorSubcoreMesh(core_axis_name="c", subcore_axis_name="s")
    fn = pl.kernel(
        functools.partial(_sc_body, nchunk=nchunk, max_iters=max_iters),
        mesh=mesh,
        out_type=jax.ShapeDtypeStruct((NC, N_PAD, H), jnp.float32),
        scratch_types=[
            pltpu.VMEM((max_iters * CHUNK,), jnp.int32),
            pltpu.VMEM((CHUNK,), jnp.int32),
            pltpu.VMEM((CHUNK,), jnp.int32),
            pltpu.VMEM((CHUNK, H), jnp.float32),
            pltpu.VMEM((CHUNK, H), jnp.float32),
            pltpu.VMEM((CHUNK, H), jnp.float32),
            pltpu.VMEM((CHUNK, H), jnp.float32),
            pltpu.VMEM_SHARED((N_PAD, H), jnp.float32),
            pltpu.SemaphoreType.DMA,
            pltpu.SemaphoreType.DMA,
            pltpu.SemaphoreType.DMA,
            pltpu.SemaphoreType.DMA,
            pltpu.SemaphoreType.DMA,
            pltpu.SemaphoreType.DMA,
        ],
    )
    return fn(h, src_half, dst_half, w, zeros)


def _f_body(p_ref, q_ref, x_ref, w2_ref, b2_ref, w3_ref, b3_ref, o_ref):
    t = (p_ref[0] + p_ref[1]) + (q_ref[0] + q_ref[1])
    t = jnp.dot(t, w2_ref[...], preferred_element_type=jnp.float32) + b2_ref[...]
    t = _ssp(t)
    o_ref[...] = (
        jnp.dot(t, w3_ref[...], preferred_element_type=jnp.float32)
        + b3_ref[...]
        + x_ref[...]
    )


def _tail(partials_a, partials_b, x, W2, b2, W3, b3):
    blk = 1000
    return pl.pallas_call(
        _f_body,
        grid=(N // blk,),
        in_specs=[
            pl.BlockSpec((NC, blk, H), lambda i: (0, i, 0)),
            pl.BlockSpec((NC, blk, H), lambda i: (0, i, 0)),
            pl.BlockSpec((blk, H), lambda i: (i, 0)),
            pl.BlockSpec((H, H), lambda i: (0, 0)),
            pl.BlockSpec((1, H), lambda i: (0, 0)),
            pl.BlockSpec((H, H), lambda i: (0, 0)),
            pl.BlockSpec((1, H), lambda i: (0, 0)),
        ],
        out_specs=pl.BlockSpec((blk, H), lambda i: (i, 0)),
        out_shape=jax.ShapeDtypeStruct((N, H), jnp.float32),
    )(partials_a, partials_b, x, W2, b2.reshape(1, H), W3, b3.reshape(1, H))


E2 = E // 2
NCHUNK2 = E2 // CHUNK
MAX_ITERS2 = -(-NCHUNK2 // (NC * NS))


def kernel(x, edge_index, rbf, W1, b1, Wf1, bf1, Wf2, bf2, W2, b2, W3, b3):
    h = _atomwise1(x, W1, b1)
    zeros = jnp.zeros((ROWS_PER_TILE, H), jnp.float32)
    src = edge_index[0]
    dst = edge_index[1]
    w_a = _filter_net(rbf[:E2], Wf1, bf1, Wf2, bf2)
    p_a = _cfconv_scatter(h, src[:E2], dst[:E2], w_a, zeros,
                          NCHUNK2, MAX_ITERS2)
    w_b = _filter_net(rbf[E2:], Wf1, bf1, Wf2, bf2)
    p_b = _cfconv_scatter(h, src[E2:], dst[E2:], w_b, zeros,
                          NCHUNK2, MAX_ITERS2)
    return _tail(p_a, p_b, x, W2, b2, W3, b3)


# speedup vs baseline: 1.0953x; 1.0953x over previous
import jax
import jax.numpy as jnp
import numpy as _np
from jax import lax
from jax.experimental import pallas as pl
from jax.experimental.pallas import tpu as pltpu
from jax.experimental.pallas import tpu_sc as plsc

N = 10000
E = 320000
H = 128
R = 128

LOG2 = 0.6931471805599453

NC = 2
NS = 16
CHUNK = 64
NCHUNK = E // CHUNK
MAX_ITERS = -(-NCHUNK // (NC * NS))
N_PAD = 10240
ROWS_PER_TILE = N_PAD // NS

def _ssp(v):
    return jnp.maximum(v, 0.0) + jnp.log(1.0 + jnp.exp(-jnp.abs(v))) - LOG2


def _h_body(x_ref, w_ref, b_ref, o_ref):
    o_ref[...] = (
        jnp.dot(x_ref[...], w_ref[...], preferred_element_type=jnp.float32)
        + b_ref[...]
    )


def _atomwise1(x, W1, b1):
    blk = 1000
    return pl.pallas_call(
        _h_body,
        grid=(N // blk,),
        in_specs=[
            pl.BlockSpec((blk, H), lambda i: (i, 0)),
            pl.BlockSpec((H, H), lambda i: (0, 0)),
            pl.BlockSpec((1, H), lambda i: (0, 0)),
        ],
        out_specs=pl.BlockSpec((blk, H), lambda i: (i, 0)),
        out_shape=jax.ShapeDtypeStruct((N, H), jnp.float32),
    )(x, W1, b1.reshape(1, H))


def _w_body(rbf_ref, wf1_ref, bf1_ref, wf2_ref, bf2_ref, o_ref):
    t = (
        jnp.dot(rbf_ref[...], wf1_ref[...], preferred_element_type=jnp.float32)
        + bf1_ref[...]
    )
    t = _ssp(t)
    o_ref[...] = (
        jnp.dot(t, wf2_ref[...], preferred_element_type=jnp.float32)
        + bf2_ref[...]
    )


def _filter_net(rbf, Wf1, bf1, Wf2, bf2):
    blk = 2000
    return pl.pallas_call(
        _w_body,
        grid=(E // blk,),
        in_specs=[
            pl.BlockSpec((blk, R), lambda i: (i, 0)),
            pl.BlockSpec((R, H), lambda i: (0, 0)),
            pl.BlockSpec((1, H), lambda i: (0, 0)),
            pl.BlockSpec((H, H), lambda i: (0, 0)),
            pl.BlockSpec((1, H), lambda i: (0, 0)),
        ],
        out_specs=pl.BlockSpec((blk, H), lambda i: (i, 0)),
        out_shape=jax.ShapeDtypeStruct((E, H), jnp.float32),
    )(rbf, Wf1, bf1.reshape(1, H), Wf2, bf2.reshape(1, H))


def _sc_body(h_hbm, src_hbm, dst_hbm, w_hbm, zeros_hbm, out_hbm,
             sslab, didx0, didx1, hrows0, hrows1, wrows0, wrows1, acc,
             sem_d0, sem_d1, sem_g0, sem_g1, sem_w0, sem_w1):
    c_id = lax.axis_index("c")
    s_id = lax.axis_index("s")
    g = c_id * NS + s_id
    c0 = (NCHUNK * g) // (NC * NS)
    cnt = (NCHUNK * (g + 1)) // (NC * NS) - c0

    bufs = (
        (didx0, hrows0, wrows0, sem_d0, sem_g0, sem_w0),
        (didx1, hrows1, wrows1, sem_d1, sem_g1, sem_w1),
    )

    pltpu.sync_copy(zeros_hbm, acc.at[pl.ds(s_id * ROWS_PER_TILE, ROWS_PER_TILE)])
    pltpu.sync_copy(src_hbm.at[pl.ds(c0 * CHUNK, MAX_ITERS * CHUNK)], sslab)
    plsc.subcore_barrier()

    def issue(i, b):
        didx, hrows, wrows, sem_d, sem_g, sem_w = bufs[b]
        pltpu.async_copy(dst_hbm.at[pl.ds((c0 + i) * CHUNK, CHUNK)], didx, sem_d)
        pltpu.async_copy(w_hbm.at[pl.ds((c0 + i) * CHUNK, CHUNK)], wrows, sem_w)
        pltpu.async_copy(h_hbm.at[sslab.at[pl.ds(i * CHUNK, CHUNK)]], hrows, sem_g)

    def wait_issue(i, b):
        didx, hrows, wrows, sem_d, sem_g, sem_w = bufs[b]
        pltpu.make_async_copy(
            dst_hbm.at[pl.ds((c0 + i) * CHUNK, CHUNK)], didx, sem_d).wait()
        pltpu.make_async_copy(
            w_hbm.at[pl.ds((c0 + i) * CHUNK, CHUNK)], wrows, sem_w).wait()
        pltpu.make_async_copy(
            h_hbm.at[sslab.at[pl.ds(i * CHUNK, CHUNK)]], hrows, sem_g).wait()

    issue(0, 0)

    def outer(k, carry):
        for b in range(2):
            i = k * 2 + b
            didx, hrows, wrows, _, _, _ = bufs[b]

            @pl.when(i + 1 < cnt)
            def _():
                issue(i + 1, 1 - b)

            @pl.when(i < cnt)
            def _():
                wait_issue(i, b)

                def mul_row(r, carry2):
                    for q in range(H // 16):
                        sl = pl.ds(q * 16, 16)
                        hrows[r, sl] = hrows[r, sl] * wrows[r, sl]
                    return carry2

                lax.fori_loop(0, CHUNK, mul_row, 0)
                pltpu.sync_copy(hrows, acc.at[didx], add=True)

        return carry

    lax.fori_loop(0, (MAX_ITERS + 1) // 2, outer, 0)
    plsc.subcore_barrier()

    pltpu.sync_copy(
        acc.at[pl.ds(s_id * ROWS_PER_TILE, ROWS_PER_TILE)],
        out_hbm.at[c_id, pl.ds(s_id * ROWS_PER_TILE, ROWS_PER_TILE)],
    )


def _cfconv_scatter(h, src_pad, dst_pad, w, zeros):
    mesh = plsc.VectorSubcoreMesh(core_axis_name="c", subcore_axis_name="s")
    fn = pl.kernel(
        _sc_body,
        mesh=mesh,
        out_type=jax.ShapeDtypeStruct((NC, N_PAD, H), jnp.float32),
        scratch_types=[
            pltpu.VMEM((MAX_ITERS * CHUNK,), jnp.int32),
            pltpu.VMEM((CHUNK,), jnp.int32),
            pltpu.VMEM((CHUNK,), jnp.int32),
            pltpu.VMEM((CHUNK, H), jnp.float32),
            pltpu.VMEM((CHUNK, H), jnp.float32),
            pltpu.VMEM((CHUNK, H), jnp.float32),
            pltpu.VMEM((CHUNK, H), jnp.float32),
            pltpu.VMEM_SHARED((N_PAD, H), jnp.float32),
            pltpu.SemaphoreType.DMA,
            pltpu.SemaphoreType.DMA,
            pltpu.SemaphoreType.DMA,
            pltpu.SemaphoreType.DMA,
            pltpu.SemaphoreType.DMA,
            pltpu.SemaphoreType.DMA,
        ],
    )
    return fn(h, src_pad, dst_pad, w, zeros)


def _f_body(p_ref, x_ref, w2_ref, b2_ref, w3_ref, b3_ref, o_ref):
    t = p_ref[0] + p_ref[1]
    t = jnp.dot(t, w2_ref[...], preferred_element_type=jnp.float32) + b2_ref[...]
    t = _ssp(t)
    o_ref[...] = (
        jnp.dot(t, w3_ref[...], preferred_element_type=jnp.float32)
        + b3_ref[...]
        + x_ref[...]
    )


def _tail(partials, x, W2, b2, W3, b3):
    blk = 1000
    return pl.pallas_call(
        _f_body,
        grid=(N // blk,),
        in_specs=[
            pl.BlockSpec((NC, blk, H), lambda i: (0, i, 0)),
            pl.BlockSpec((blk, H), lambda i: (i, 0)),
            pl.BlockSpec((H, H), lambda i: (0, 0)),
            pl.BlockSpec((1, H), lambda i: (0, 0)),
            pl.BlockSpec((H, H), lambda i: (0, 0)),
            pl.BlockSpec((1, H), lambda i: (0, 0)),
        ],
        out_specs=pl.BlockSpec((blk, H), lambda i: (i, 0)),
        out_shape=jax.ShapeDtypeStruct((N, H), jnp.float32),
    )(partials, x, W2, b2.reshape(1, H), W3, b3.reshape(1, H))


def kernel(x, edge_index, rbf, W1, b1, Wf1, bf1, Wf2, bf2, W2, b2, W3, b3):
    h = _atomwise1(x, W1, b1)
    w = _filter_net(rbf, Wf1, bf1, Wf2, bf2)
    zeros = jnp.zeros((ROWS_PER_TILE, H), jnp.float32)
    partials = _cfconv_scatter(h, edge_index[0], edge_index[1], w, zeros)
    return _tail(partials, x, W2, b2, W3, b3)
